# baseline (speedup 1.0000x reference)
import jax
import jax.numpy as jnp
from jax.experimental import pallas as pl
from jax.experimental.pallas import tpu as pltpu


def _copy_body(xs_ref, xt_ref, ea_ref, u_ref,
               oxs_ref, oxt_ref, oea_ref, ou_ref, sem):
    copies = [
        pltpu.make_async_copy(xs_ref, oxs_ref, sem.at[0]),
        pltpu.make_async_copy(xt_ref, oxt_ref, sem.at[1]),
        pltpu.make_async_copy(ea_ref, oea_ref, sem.at[2]),
        pltpu.make_async_copy(u_ref, ou_ref, sem.at[3]),
    ]
    for c in copies:
        c.start()
    for c in copies:
        c.wait()


def kernel(x_s, x_t, edge_index, edge_attr, u, batch_e, batch_s, batch_t):
    del edge_index, batch_e, batch_s, batch_t
    hbm = pl.BlockSpec(memory_space=pltpu.MemorySpace.HBM)
    outs = pl.pallas_call(
        _copy_body,
        in_specs=[hbm, hbm, hbm, hbm],
        out_specs=[hbm, hbm, hbm, hbm],
        out_shape=[
            jax.ShapeDtypeStruct(x_s.shape, x_s.dtype),
            jax.ShapeDtypeStruct(x_t.shape, x_t.dtype),
            jax.ShapeDtypeStruct(edge_attr.shape, edge_attr.dtype),
            jax.ShapeDtypeStruct(u.shape, u.dtype),
        ],
        scratch_shapes=[pltpu.SemaphoreType.DMA((4,))],
    )(x_s, x_t, edge_attr, u)
    return tuple(outs)


# speedup vs baseline: 21.4803x; 21.4803x over previous
import jax
import jax.numpy as jnp
from jax.experimental import pallas as pl

_GRID = 10


def _copy_body(xs_ref, xt_ref, ea_ref, u_ref, oxs_ref, oxt_ref, oea_ref, ou_ref):
    oxs_ref[...] = xs_ref[...]
    oxt_ref[...] = xt_ref[...]
    oea_ref[...] = ea_ref[...]

    @pl.when(pl.program_id(0) == 0)
    def _():
        ou_ref[...] = u_ref[...]


def kernel(x_s, x_t, edge_index, edge_attr, u, batch_e, batch_s, batch_t):
    del edge_index, batch_e, batch_s, batch_t
    n_s, d_feat = x_s.shape
    e, d_edge = edge_attr.shape
    bx = n_s // _GRID
    be = e // _GRID

    specs = [
        pl.BlockSpec((bx, d_feat), lambda i: (i, 0)),
        pl.BlockSpec((bx, d_feat), lambda i: (i, 0)),
        pl.BlockSpec((be, d_edge), lambda i: (i, 0)),
        pl.BlockSpec(u.shape, lambda i: (0, 0)),
    ]
    outs = pl.pallas_call(
        _copy_body,
        grid=(_GRID,),
        in_specs=specs,
        out_specs=specs,
        out_shape=[
            jax.ShapeDtypeStruct(x_s.shape, x_s.dtype),
            jax.ShapeDtypeStruct(x_t.shape, x_t.dtype),
            jax.ShapeDtypeStruct(edge_attr.shape, edge_attr.dtype),
            jax.ShapeDtypeStruct(u.shape, u.dtype),
        ],
    )(x_s, x_t, edge_attr, u)
    return tuple(outs)
